# baseline (speedup 1.0000x reference)
import jax
from jax.experimental import pallas as pl
from jax.experimental.pallas import tpu as pltpu


_TC_STEPS = 50


def _copy3_kernel(u_ref, i_ref, t_ref, ou_ref, oi_ref, ot_ref):
    ou_ref[...] = u_ref[...]
    oi_ref[...] = i_ref[...]
    ot_ref[...] = t_ref[...]


def kernel(embed_user, embed_item, embed_tag):
    nu, d = embed_user.shape
    ni, _ = embed_item.shape
    nt, _ = embed_tag.shape
    bu, bi, bt = nu // _TC_STEPS, ni // _TC_STEPS, nt // _TC_STEPS

    def spec(block_rows):
        return pl.BlockSpec((block_rows, d), lambda s: (s, 0))

    return pl.pallas_call(
        _copy3_kernel,
        grid=(_TC_STEPS,),
        compiler_params=pltpu.CompilerParams(dimension_semantics=("parallel",)),
        in_specs=[spec(bu), spec(bi), spec(bt)],
        out_specs=[spec(bu), spec(bi), spec(bt)],
        out_shape=[
            jax.ShapeDtypeStruct(embed_user.shape, embed_user.dtype),
            jax.ShapeDtypeStruct(embed_item.shape, embed_item.dtype),
            jax.ShapeDtypeStruct(embed_tag.shape, embed_tag.dtype),
        ],
    )(embed_user, embed_item, embed_tag)


# speedup vs baseline: 1.0762x; 1.0762x over previous
import jax
from jax.experimental import pallas as pl
from jax.experimental.pallas import tpu as pltpu


_TC_STEPS = 10


def _copy3_kernel(u_ref, i_ref, t_ref, ou_ref, oi_ref, ot_ref):
    ou_ref[...] = u_ref[...]
    oi_ref[...] = i_ref[...]
    ot_ref[...] = t_ref[...]


def kernel(embed_user, embed_item, embed_tag):
    nu, d = embed_user.shape
    ni, _ = embed_item.shape
    nt, _ = embed_tag.shape
    bu, bi, bt = nu // _TC_STEPS, ni // _TC_STEPS, nt // _TC_STEPS

    def spec(block_rows):
        return pl.BlockSpec((block_rows, d), lambda s: (s, 0))

    return pl.pallas_call(
        _copy3_kernel,
        grid=(_TC_STEPS,),
        compiler_params=pltpu.CompilerParams(dimension_semantics=("parallel",)),
        in_specs=[spec(bu), spec(bi), spec(bt)],
        out_specs=[spec(bu), spec(bi), spec(bt)],
        out_shape=[
            jax.ShapeDtypeStruct(embed_user.shape, embed_user.dtype),
            jax.ShapeDtypeStruct(embed_item.shape, embed_item.dtype),
            jax.ShapeDtypeStruct(embed_tag.shape, embed_tag.dtype),
        ],
    )(embed_user, embed_item, embed_tag)
